# Initial kernel scaffold; baseline (speedup 1.0000x reference)
import jax
import jax.numpy as jnp
from jax.experimental import pallas as pl


def kernel(x, score, sigma, group_id, noise):
    raise NotImplementedError("write your pallas kernel here")


# speedup vs baseline: 49.4497x; 49.4497x over previous
import numpy as np
import jax
import jax.numpy as jnp
from jax import lax
from jax.experimental import pallas as pl

_NS = 500
_NR = 49
_CC = 32


def _pool_matrix():
    m = np.zeros((196, _NR), np.float32)
    for r in range(_NR):
        ri, rj = r // 7, r % 7
        for ki in range(2):
            for kj in range(2):
                m[(2 * ri + ki) * 14 + (2 * rj + kj), r] = 1.0
    return m


def _indicator_body(score_ref, m_ref, sig_ref, noise_ref, gid_ref, w_ref):
    pool = lax.dot_general(
        score_ref[...], m_ref[...], (((1,), (0,)), ((), ())),
        preferred_element_type=jnp.float32,
        precision=lax.Precision.HIGHEST) * 0.25
    mn = jnp.min(pool, axis=1, keepdims=True)
    mx = jnp.max(pool, axis=1, keepdims=True)
    sn = (pool - mn) / (mx - mn + 1e-5)
    sig = sig_ref[0, 0]
    pert = sn[:, None, :] + noise_ref[...] * sig
    mxv = jnp.max(pert, axis=2, keepdims=True)
    iota = lax.broadcasted_iota(jnp.int32, (8, _NS, _NR), 2)
    cand = jnp.where(pert == mxv, iota, _NR)
    idx = jnp.min(cand, axis=2, keepdims=True)
    onehot = (iota == idx).astype(jnp.float32)
    hist = jnp.sum(onehot, axis=1) / np.float32(_NS)
    hist3 = hist.reshape(2, 4, _NR)
    gid = gid_ref[...]
    keys = lax.broadcasted_iota(jnp.int32, (2, 4, 4), 2)
    mask = (gid[:, :, None] == keys).astype(jnp.float32)
    w = jnp.sum(mask[:, :, :, None] * hist3[:, None, :, :], axis=2)
    w_ref[...] = w.reshape(2, 4, 1, _NR)


def _mix_body(w_ref, x_ref, o_ref):
    xb = x_ref[0, :, 0]
    row = lax.broadcasted_iota(jnp.int32, (224, 32), 0)
    col = lax.broadcasted_iota(jnp.int32, (224, 32), 1)
    acc = jnp.zeros((_CC, 32, 32), jnp.float32)
    for ri in range(7):
        b = jnp.zeros((224, 32), jnp.float32)
        for rj in range(7):
            b = b + jnp.where(row == col + 32 * rj,
                              w_ref[0, 0, 0, ri * 7 + rj], 0.0)
        xs = xb[:, ri * 32:(ri + 1) * 32, :].reshape(_CC * 32, 224)
        acc = acc + lax.dot_general(
            xs, b, (((1,), (0,)), ((), ())),
            preferred_element_type=jnp.float32,
            precision=lax.Precision.HIGHEST).reshape(_CC, 32, 32)
    o_ref[0, 0] = acc


def _compute_w(score, sigma, group_id, noise, interpret=False):
    score_flat = score.reshape(8, 196)
    m = jnp.asarray(_pool_matrix())
    sig = jnp.reshape(sigma, (1, 1)).astype(jnp.float32)
    return pl.pallas_call(
        _indicator_body,
        out_shape=jax.ShapeDtypeStruct((2, 4, 1, _NR), jnp.float32),
        interpret=interpret,
    )(score_flat, m, sig, noise, group_id)


def _mix(x, w, interpret=False):
    b, c, t = 2, 96, 4
    out = pl.pallas_call(
        _mix_body,
        grid=(b, t, c // _CC),
        in_specs=[
            pl.BlockSpec((1, 1, 1, _NR), lambda bi, ti, ci: (bi, ti, 0, 0)),
            pl.BlockSpec((1, _CC, 1, 224, 224),
                         lambda bi, ti, ci: (bi, ci, ti, 0, 0)),
        ],
        out_specs=pl.BlockSpec((1, 1, _CC, 32, 32),
                               lambda bi, ti, ci: (bi, ti, ci, 0, 0)),
        out_shape=jax.ShapeDtypeStruct((b, t, c, 32, 32), jnp.float32),
        interpret=interpret,
    )(w, x)
    out = out.reshape(b, t, c, 2, 16, 2, 16)
    out = jnp.transpose(out, (0, 1, 2, 4, 6, 3, 5))
    return out.reshape(b * t, 1, c * 1024)


def kernel(x, score, sigma, group_id, noise):
    w = _compute_w(score, sigma, group_id, noise)
    return _mix(x, w)


# speedup vs baseline: 60.5553x; 1.2246x over previous
import numpy as np
import jax
import jax.numpy as jnp
from jax import lax
from jax.experimental import pallas as pl

_NS = 500
_NR = 49
_CC = 32


def _pool_matrix():
    m = np.zeros((196, _NR), np.float32)
    for r in range(_NR):
        ri, rj = r // 7, r % 7
        for ki in range(2):
            for kj in range(2):
                m[(2 * ri + ki) * 14 + (2 * rj + kj), r] = 1.0
    return m


def _indicator_body(score_ref, m_ref, sig_ref, noise_ref, gid_ref, w_ref):
    pool = lax.dot_general(
        score_ref[...], m_ref[...], (((1,), (0,)), ((), ())),
        preferred_element_type=jnp.float32,
        precision=lax.Precision.HIGHEST) * 0.25
    mn = jnp.min(pool, axis=1, keepdims=True)
    mx = jnp.max(pool, axis=1, keepdims=True)
    sn = (pool - mn) / (mx - mn + 1e-5)
    sig = sig_ref[0, 0]
    pert = sn[:, None, :] + noise_ref[...] * sig
    mxv = jnp.max(pert, axis=2, keepdims=True)
    iota = lax.broadcasted_iota(jnp.int32, (8, _NS, _NR), 2)
    cand = jnp.where(pert == mxv, iota, _NR)
    idx = jnp.min(cand, axis=2, keepdims=True)
    onehot = (iota == idx).astype(jnp.float32)
    hist = jnp.sum(onehot, axis=1) / np.float32(_NS)
    hist3 = hist.reshape(2, 4, _NR)
    gid = gid_ref[...]
    keys = lax.broadcasted_iota(jnp.int32, (2, 4, 4), 2)
    mask = (gid[:, :, None] == keys).astype(jnp.float32)
    w = jnp.sum(mask[:, :, :, None] * hist3[:, None, :, :], axis=2)
    w_ref[...] = w.reshape(2, 4, 1, _NR)


def _mix_body(w_ref, x_ref, o_ref):
    xb = x_ref[0, :, 0]
    row = lax.broadcasted_iota(jnp.int32, (224, 32), 0)
    col = lax.broadcasted_iota(jnp.int32, (224, 32), 1)
    acc = jnp.zeros((_CC, 32, 32), jnp.float32)
    for ri in range(7):
        b = jnp.zeros((224, 32), jnp.float32)
        for rj in range(7):
            b = b + jnp.where(row == col + 32 * rj,
                              w_ref[0, 0, 0, ri * 7 + rj], 0.0)
        xs = xb[:, ri * 32:(ri + 1) * 32, :].reshape(_CC * 32, 224)
        acc = acc + lax.dot_general(
            xs, b, (((1,), (0,)), ((), ())),
            preferred_element_type=jnp.float32).reshape(_CC, 32, 32)
    o_ref[0, 0] = acc


def _compute_w(score, sigma, group_id, noise, interpret=False):
    score_flat = score.reshape(8, 196)
    m = jnp.asarray(_pool_matrix())
    sig = jnp.reshape(sigma, (1, 1)).astype(jnp.float32)
    return pl.pallas_call(
        _indicator_body,
        out_shape=jax.ShapeDtypeStruct((2, 4, 1, _NR), jnp.float32),
        interpret=interpret,
    )(score_flat, m, sig, noise, group_id)


def _mix(x, w, interpret=False):
    b, c, t = 2, 96, 4
    out = pl.pallas_call(
        _mix_body,
        grid=(b, t, c // _CC),
        in_specs=[
            pl.BlockSpec((1, 1, 1, _NR), lambda bi, ti, ci: (bi, ti, 0, 0)),
            pl.BlockSpec((1, _CC, 1, 224, 224),
                         lambda bi, ti, ci: (bi, ci, ti, 0, 0)),
        ],
        out_specs=pl.BlockSpec((1, 1, _CC, 32, 32),
                               lambda bi, ti, ci: (bi, ti, ci, 0, 0)),
        out_shape=jax.ShapeDtypeStruct((b, t, c, 32, 32), jnp.float32),
        interpret=interpret,
    )(w, x)
    out = out.reshape(b, t, c, 2, 16, 2, 16)
    out = jnp.transpose(out, (0, 1, 2, 4, 6, 3, 5))
    return out.reshape(b * t, 1, c * 1024)


def kernel(x, score, sigma, group_id, noise):
    w = _compute_w(score, sigma, group_id, noise)
    return _mix(x, w)


# speedup vs baseline: 317.2654x; 5.2393x over previous
import numpy as np
import jax
import jax.numpy as jnp
from jax import lax
from jax.experimental import pallas as pl

_NS = 500
_NR = 49
_CC = 32


def _pool_matrix():
    m = np.zeros((196, _NR), np.float32)
    for r in range(_NR):
        ri, rj = r // 7, r % 7
        for ki in range(2):
            for kj in range(2):
                m[(2 * ri + ki) * 14 + (2 * rj + kj), r] = 1.0
    return m


def _indicator_body(score_ref, m_ref, sig_ref, noise_ref, gid_ref, w_ref):
    pool = lax.dot_general(
        score_ref[...], m_ref[...], (((1,), (0,)), ((), ())),
        preferred_element_type=jnp.float32,
        precision=lax.Precision.HIGHEST) * 0.25
    mn = jnp.min(pool, axis=1, keepdims=True)
    mx = jnp.max(pool, axis=1, keepdims=True)
    sn = (pool - mn) / (mx - mn + 1e-5)
    sig = sig_ref[0, 0]
    pert = sn[:, None, :] + noise_ref[...] * sig
    mxv = jnp.max(pert, axis=2, keepdims=True)
    iota = lax.broadcasted_iota(jnp.int32, (8, _NS, _NR), 2)
    cand = jnp.where(pert == mxv, iota, _NR)
    idx = jnp.min(cand, axis=2, keepdims=True)
    onehot = (iota == idx).astype(jnp.float32)
    hist = jnp.sum(onehot, axis=1) / np.float32(_NS)
    hist3 = hist.reshape(2, 4, _NR)
    gid = gid_ref[...]
    keys = lax.broadcasted_iota(jnp.int32, (2, 4, 4), 2)
    mask = (gid[:, :, None] == keys).astype(jnp.float32)
    w = jnp.sum(mask[:, :, :, None] * hist3[:, None, :, :], axis=2)
    w_ref[...] = w.reshape(2, 4, 1, _NR)


def _mix_body(w_ref, x_ref, o_ref):
    xb = x_ref[0, :, 0]
    wp = lax.broadcasted_iota(jnp.int32, (224, 64), 0)
    q = lax.broadcasted_iota(jnp.int32, (224, 64), 1)
    rj = wp // 32
    kj = (wp % 32) // 16
    aj = wp % 16
    ajq = q // 4
    kiq = (q // 2) % 2
    kjq = q % 2
    colmask = (ajq == aj) & (kjq == kj)
    kimask = [colmask & (kiq == 0), colmask & (kiq == 1)]
    acc = jnp.zeros((_CC * 16, 64), jnp.float32)
    for ri in range(7):
        wsel = jnp.zeros((224, 64), jnp.float32)
        for j in range(7):
            wsel = wsel + jnp.where(rj == j, w_ref[0, 0, 0, ri * 7 + j], 0.0)
        for ki in range(2):
            b = jnp.where(kimask[ki], wsel, 0.0)
            base = ri * 32 + ki * 16
            xs = xb[:, base:base + 16, :].reshape(_CC * 16, 224)
            acc = acc + lax.dot_general(
                xs, b, (((1,), (0,)), ((), ())),
                preferred_element_type=jnp.float32)
    o_ref[0, 0] = acc.reshape(_CC, 16, 64)


def _compute_w(score, sigma, group_id, noise, interpret=False):
    score_flat = score.reshape(8, 196)
    m = jnp.asarray(_pool_matrix())
    sig = jnp.reshape(sigma, (1, 1)).astype(jnp.float32)
    return pl.pallas_call(
        _indicator_body,
        out_shape=jax.ShapeDtypeStruct((2, 4, 1, _NR), jnp.float32),
        interpret=interpret,
    )(score_flat, m, sig, noise, group_id)


def _mix(x, w, interpret=False):
    b, c, t = 2, 96, 4
    out = pl.pallas_call(
        _mix_body,
        grid=(b, t, c // _CC),
        in_specs=[
            pl.BlockSpec((1, 1, 1, _NR), lambda bi, ti, ci: (bi, ti, 0, 0)),
            pl.BlockSpec((1, _CC, 1, 224, 224),
                         lambda bi, ti, ci: (bi, ci, ti, 0, 0)),
        ],
        out_specs=pl.BlockSpec((1, 1, _CC, 16, 64),
                               lambda bi, ti, ci: (bi, ti, ci, 0, 0)),
        out_shape=jax.ShapeDtypeStruct((b, t, c, 16, 64), jnp.float32),
        interpret=interpret,
    )(w, x)
    return out.reshape(b * t, 1, c * 1024)


def kernel(x, score, sigma, group_id, noise):
    w = _compute_w(score, sigma, group_id, noise)
    return _mix(x, w)


# speedup vs baseline: 328.4649x; 1.0353x over previous
import numpy as np
import jax
import jax.numpy as jnp
from jax import lax
from jax.experimental import pallas as pl

_NS = 500
_NR = 49
_CC = 48


def _pool_matrix():
    m = np.zeros((196, _NR), np.float32)
    for r in range(_NR):
        ri, rj = r // 7, r % 7
        for ki in range(2):
            for kj in range(2):
                m[(2 * ri + ki) * 14 + (2 * rj + kj), r] = 1.0
    return m


def _indicator_body(score_ref, m_ref, sig_ref, noise_ref, gid_ref, w_ref):
    pool = lax.dot_general(
        score_ref[...], m_ref[...], (((1,), (0,)), ((), ())),
        preferred_element_type=jnp.float32,
        precision=lax.Precision.HIGHEST) * 0.25
    mn = jnp.min(pool, axis=1, keepdims=True)
    mx = jnp.max(pool, axis=1, keepdims=True)
    sn = (pool - mn) / (mx - mn + 1e-5)
    sig = sig_ref[0, 0]
    pert = sn[:, None, :] + noise_ref[...] * sig
    mxv = jnp.max(pert, axis=2, keepdims=True)
    iota = lax.broadcasted_iota(jnp.int32, (8, _NS, _NR), 2)
    cand = jnp.where(pert == mxv, iota, _NR)
    idx = jnp.min(cand, axis=2, keepdims=True)
    onehot = (iota == idx).astype(jnp.float32)
    hist = jnp.sum(onehot, axis=1) / np.float32(_NS)
    hist3 = hist.reshape(2, 4, _NR)
    gid = gid_ref[...]
    keys = lax.broadcasted_iota(jnp.int32, (2, 4, 4), 2)
    mask = (gid[:, :, None] == keys).astype(jnp.float32)
    w = jnp.sum(mask[:, :, :, None] * hist3[:, None, :, :], axis=2)
    w_ref[...] = w.reshape(2, 4, 1, _NR)


def _mix_body(w_ref, x_ref, o_ref):
    xb = x_ref[0, :, 0]
    wp = lax.broadcasted_iota(jnp.int32, (224, 64), 0)
    q = lax.broadcasted_iota(jnp.int32, (224, 64), 1)
    rj = wp // 32
    kj = (wp % 32) // 16
    aj = wp % 16
    ajq = q // 4
    kiq = (q // 2) % 2
    kjq = q % 2
    colmask = (ajq == aj) & (kjq == kj)
    kimask = [colmask & (kiq == 0), colmask & (kiq == 1)]
    acc = jnp.zeros((_CC * 16, 64), jnp.float32)
    for ri in range(7):
        wsel = jnp.zeros((224, 64), jnp.float32)
        for j in range(7):
            wsel = wsel + jnp.where(rj == j, w_ref[0, 0, 0, ri * 7 + j], 0.0)
        for ki in range(2):
            b = jnp.where(kimask[ki], wsel, 0.0)
            base = ri * 32 + ki * 16
            xs = xb[:, base:base + 16, :].reshape(_CC * 16, 224)
            acc = acc + lax.dot_general(
                xs, b, (((1,), (0,)), ((), ())),
                preferred_element_type=jnp.float32)
    o_ref[0, 0] = acc.reshape(_CC, 16, 64)


def _compute_w(score, sigma, group_id, noise, interpret=False):
    score_flat = score.reshape(8, 196)
    m = jnp.asarray(_pool_matrix())
    sig = jnp.reshape(sigma, (1, 1)).astype(jnp.float32)
    return pl.pallas_call(
        _indicator_body,
        out_shape=jax.ShapeDtypeStruct((2, 4, 1, _NR), jnp.float32),
        interpret=interpret,
    )(score_flat, m, sig, noise, group_id)


def _mix(x, w, interpret=False):
    b, c, t = 2, 96, 4
    out = pl.pallas_call(
        _mix_body,
        grid=(b, t, c // _CC),
        in_specs=[
            pl.BlockSpec((1, 1, 1, _NR), lambda bi, ti, ci: (bi, ti, 0, 0)),
            pl.BlockSpec((1, _CC, 1, 224, 224),
                         lambda bi, ti, ci: (bi, ci, ti, 0, 0)),
        ],
        out_specs=pl.BlockSpec((1, 1, _CC, 16, 64),
                               lambda bi, ti, ci: (bi, ti, ci, 0, 0)),
        out_shape=jax.ShapeDtypeStruct((b, t, c, 16, 64), jnp.float32),
        interpret=interpret,
    )(w, x)
    return out.reshape(b * t, 1, c * 1024)


def kernel(x, score, sigma, group_id, noise):
    w = _compute_w(score, sigma, group_id, noise)
    return _mix(x, w)
